# baseline (speedup 1.0000x reference)
import functools

import jax
import jax.numpy as jnp
from jax import lax
from jax.experimental import pallas as pl
from jax.experimental.pallas import tpu as pltpu
from jax.experimental.pallas import tpu_sc as plsc

_EMBED_SCALE = 1.0
_LN_EPS = 1e-5
_NC, _NS, _L = 2, 16, 16
_NW = _NC * _NS


def _rne_bf16_bits(u):
    return (u + jnp.uint32(0x7FFF) + ((u >> 16) & jnp.uint32(1))) >> 16


def _ln_body(w_ref, g_ref, b_ref, o_ref):
    x = w_ref[...] * _EMBED_SCALE
    mean = jnp.mean(x, axis=-1, keepdims=True)
    xc = x - mean
    var = jnp.mean(xc * xc, axis=-1, keepdims=True)
    y = xc * lax.rsqrt(var + _LN_EPS) * g_ref[...] + b_ref[...]
    yu = lax.bitcast_convert_type(y, jnp.uint32)
    ep = yu.shape[-1] // 2
    packed = _rne_bf16_bits(yu[:, :ep]) | (_rne_bf16_bits(yu[:, ep:]) << 16)
    o_ref[...] = lax.bitcast_convert_type(packed, jnp.int32)


def _ln_table(w, gamma, beta):
    v, e = w.shape
    blk = 2048
    return pl.pallas_call(
        _ln_body,
        grid=(v // blk,),
        in_specs=[
            pl.BlockSpec((blk, e), lambda i: (i, 0)),
            pl.BlockSpec((1, e), lambda i: (0, 0)),
            pl.BlockSpec((1, e), lambda i: (0, 0)),
        ],
        out_specs=pl.BlockSpec((blk, e // 2), lambda i: (i, 0)),
        out_shape=jax.ShapeDtypeStruct((v, e // 2), jnp.int32),
    )(w, gamma.reshape(1, e), beta.reshape(1, e))


@functools.lru_cache(maxsize=None)
def _make_sc_gather(e, n_out, cs, b):
    k_per_w = n_out // _NW
    cpc = 2
    r = b * cpc
    n_chunk = k_per_w // r
    ep = e // 2
    rc = r * cs
    gpp = b * cs // _L
    mesh = plsc.VectorSubcoreMesh(core_axis_name="c", subcore_axis_name="s")

    @functools.partial(
        pl.kernel,
        mesh=mesh,
        out_type=jax.ShapeDtypeStruct((n_out, e), jnp.float32),
        compiler_params=pltpu.CompilerParams(needs_layout_passes=False),
        scratch_types=[
            pltpu.VMEM((b, n_chunk * cpc * cs), jnp.int32),
            pltpu.VMEM((2 * rc,), jnp.int32),
            pltpu.VMEM((2, rc, ep), jnp.int32),
            pltpu.VMEM((r, e), jnp.float32),
            pltpu.SemaphoreType.DMA,
            pltpu.SemaphoreType.DMA,
            pltpu.SemaphoreType.DMA,
        ],
    )
    def sc_gather(tab_hbm, ids_hbm, out_hbm, idx2_v, idxc_v, rows_v, acc_v,
                  gsem0, gsem1, osem):
        wid = lax.axis_index("s") * _NC + lax.axis_index("c")
        base = wid * k_per_w
        gsems = (gsem0, gsem1)

        ncols = n_chunk * cpc * cs
        pltpu.sync_copy(ids_hbm.at[:, pl.ds(wid * ncols, ncols)], idx2_v)

        lane = lax.iota(jnp.int32, _L)
        rvec = lane >> 2
        svec = lane & 3

        def build_idx(slot, i):
            for v4 in range(rc // _L):
                p, v4p = divmod(v4, gpp)
                cvec = svec + cs * (cpc * i + p)
                g = plsc.load_gather(
                    idx2_v, [rvec + (_L // cs) * v4p, cvec])
                idxc_v[pl.ds(slot * rc + _L * v4, _L)] = g

        def gather_cp(slot, i):
            return pltpu.make_async_copy(
                tab_hbm.at[idxc_v.at[pl.ds(slot * rc, rc)]],
                rows_v.at[slot], gsems[slot])

        def out_cp(i):
            return pltpu.make_async_copy(
                acc_v, out_hbm.at[pl.ds(base + i * r, r)], osem)

        def accumulate(slot):
            def row(rr, c2):
                for j in range(ep // _L):
                    o = _L * j
                    lo = None
                    hi = None
                    for s in range(cs):
                        w = rows_v[slot, cs * rr + s, pl.ds(o, _L)]
                        a = plsc.bitcast(w << 16, jnp.float32)
                        b2 = plsc.bitcast(w, jnp.float32)
                        lo = a if lo is None else lo + a
                        hi = b2 if hi is None else hi + b2
                    acc_v[rr, pl.ds(o, _L)] = lo
                    acc_v[rr, pl.ds(ep + o, _L)] = hi
                return c2
            lax.fori_loop(0, r, row, 0)

        build_idx(0, 0)
        gather_cp(0, 0).start()

        def outer(i2, carry):
            for slot in range(2):
                i = i2 * 2 + slot

                @pl.when(i + 1 < n_chunk)
                def _():
                    build_idx(1 - slot, i + 1)
                    gather_cp(1 - slot, i + 1).start()

                gather_cp(slot, i).wait()

                @pl.when(i >= 1)
                def _():
                    out_cp(i - 1).wait()

                accumulate(slot)
                out_cp(i).start()
            return carry

        lax.fori_loop(0, n_chunk // 2, outer, 0)
        out_cp(n_chunk - 1).wait()

    return sc_gather


def kernel(input_ids, embed_weight, weighted_sum, ln_gamma, ln_beta):
    del weighted_sum
    cs = 4
    b, f = input_ids.shape
    code_dim = f // cs
    v, e = embed_weight.shape
    n_out = b * code_dim

    tab_pk = _ln_table(embed_weight, ln_gamma, ln_beta)
    out_flat = _make_sc_gather(e, n_out, cs, b)(tab_pk, input_ids)
    return out_flat.reshape(b, code_dim, e)


# speedup vs baseline: 1.1227x; 1.1227x over previous
import functools

import jax
import jax.numpy as jnp
from jax import lax
from jax.experimental import pallas as pl
from jax.experimental.pallas import tpu as pltpu
from jax.experimental.pallas import tpu_sc as plsc

_EMBED_SCALE = 1.0
_LN_EPS = 1e-5
_NC, _NS, _L = 2, 16, 16
_NW = _NC * _NS


def _rne_bf16_bits(u):
    return (u + jnp.uint32(0x7FFF) + ((u >> 16) & jnp.uint32(1))) >> 16


def _ln_body(w_ref, g_ref, b_ref, o_ref):
    x = w_ref[...] * _EMBED_SCALE
    mean = jnp.mean(x, axis=-1, keepdims=True)
    xc = x - mean
    var = jnp.mean(xc * xc, axis=-1, keepdims=True)
    y = xc * lax.rsqrt(var + _LN_EPS) * g_ref[...] + b_ref[...]
    yu = lax.bitcast_convert_type(y, jnp.uint32)
    ep = yu.shape[-1] // 2
    packed = _rne_bf16_bits(yu[:, :ep]) | (_rne_bf16_bits(yu[:, ep:]) << 16)
    o_ref[...] = lax.bitcast_convert_type(packed, jnp.int32)


def _ln_table(w, gamma, beta):
    v, e = w.shape
    blk = 2048
    return pl.pallas_call(
        _ln_body,
        grid=(v // blk,),
        in_specs=[
            pl.BlockSpec((blk, e), lambda i: (i, 0)),
            pl.BlockSpec((1, e), lambda i: (0, 0)),
            pl.BlockSpec((1, e), lambda i: (0, 0)),
        ],
        out_specs=pl.BlockSpec((blk, e // 2), lambda i: (i, 0)),
        out_shape=jax.ShapeDtypeStruct((v, e // 2), jnp.int32),
    )(w, gamma.reshape(1, e), beta.reshape(1, e))


@functools.lru_cache(maxsize=None)
def _make_sc_gather(e, n_out, cs, b):
    k_per_w = n_out // _NW
    r = b
    n_chunk = k_per_w // r
    ep = e // 2
    rc = r * cs
    mesh = plsc.VectorSubcoreMesh(core_axis_name="c", subcore_axis_name="s")

    @functools.partial(
        pl.kernel,
        mesh=mesh,
        out_type=jax.ShapeDtypeStruct((n_out, e), jnp.float32),
        compiler_params=pltpu.CompilerParams(needs_layout_passes=False),
        scratch_types=[
            pltpu.VMEM((b, n_chunk * cs), jnp.int32),
            pltpu.VMEM((2 * rc,), jnp.int32),
            pltpu.VMEM((2, rc, ep), jnp.int32),
            pltpu.VMEM((2, r, e), jnp.float32),
            pltpu.SemaphoreType.DMA,
            pltpu.SemaphoreType.DMA,
            pltpu.SemaphoreType.DMA,
            pltpu.SemaphoreType.DMA,
        ],
    )
    def sc_gather(tab_hbm, ids_hbm, out_hbm, idx2_v, idxc_v, rows_v, acc_v,
                  gsem0, gsem1, osem0, osem1):
        wid = lax.axis_index("s") * _NC + lax.axis_index("c")
        base = wid * k_per_w
        gsems = (gsem0, gsem1)
        osems = (osem0, osem1)

        pltpu.sync_copy(ids_hbm.at[:, pl.ds(wid * n_chunk * cs, n_chunk * cs)],
                        idx2_v)

        lane = lax.iota(jnp.int32, _L)
        rvec = lane >> 2
        svec = lane & 3

        def build_idx(slot, i):
            cvec = svec + cs * i
            for v4 in range(rc // _L):
                g = plsc.load_gather(idx2_v, [rvec + (_L // cs) * v4, cvec])
                idxc_v[pl.ds(slot * rc + _L * v4, _L)] = g

        def gather_cp(slot, i):
            return pltpu.make_async_copy(
                tab_hbm.at[idxc_v.at[pl.ds(slot * rc, rc)]],
                rows_v.at[slot], gsems[slot])

        def out_cp(aslot, i):
            return pltpu.make_async_copy(
                acc_v.at[aslot], out_hbm.at[pl.ds(base + i * r, r)],
                osems[aslot])

        def accumulate(slot, aslot):
            def row(rr, c2):
                for j in range(ep // _L):
                    o = _L * j
                    lo = None
                    hi = None
                    for s in range(cs):
                        w = rows_v[slot, cs * rr + s, pl.ds(o, _L)]
                        a = plsc.bitcast(w << 16, jnp.float32)
                        b2 = plsc.bitcast(w, jnp.float32)
                        lo = a if lo is None else lo + a
                        hi = b2 if hi is None else hi + b2
                    acc_v[aslot, rr, pl.ds(o, _L)] = lo
                    acc_v[aslot, rr, pl.ds(ep + o, _L)] = hi
                return c2
            lax.fori_loop(0, r, row, 0)

        build_idx(0, 0)
        gather_cp(0, 0).start()

        def outer(i2, carry):
            for slot in range(2):
                i = i2 * 2 + slot

                @pl.when(i + 1 < n_chunk)
                def _():
                    build_idx(1 - slot, i + 1)
                    gather_cp(1 - slot, i + 1).start()

                gather_cp(slot, i).wait()

                @pl.when(i >= 2)
                def _():
                    out_cp(slot, i - 2).wait()

                accumulate(slot, slot)
                out_cp(slot, i).start()
            return carry

        lax.fori_loop(0, n_chunk // 2, outer, 0)
        out_cp(0, n_chunk - 2).wait()
        out_cp(1, n_chunk - 1).wait()

    return sc_gather


def kernel(input_ids, embed_weight, weighted_sum, ln_gamma, ln_beta):
    del weighted_sum
    cs = 4
    b, f = input_ids.shape
    code_dim = f // cs
    v, e = embed_weight.shape
    n_out = b * code_dim

    tab_pk = _ln_table(embed_weight, ln_gamma, ln_beta)
    out_flat = _make_sc_gather(e, n_out, cs, b)(tab_pk, input_ids)
    return out_flat.reshape(b, code_dim, e)
